# Initial kernel scaffold; baseline (speedup 1.0000x reference)
import jax
import jax.numpy as jnp
from jax.experimental import pallas as pl


def kernel(x, edge_index, edge_attr, batch, W_ec1, b_ec1, W_ec2, b_ec2, W0, b0, W1, b1, W2, b2, Wo1, bo1, Wo2, bo2):
    raise NotImplementedError("write your pallas kernel here")


# speedup vs baseline: 3.0432x; 3.0432x over previous
import functools

import jax
import jax.numpy as jnp
from jax import lax
from jax.experimental import pallas as pl
from jax.experimental.pallas import tpu as pltpu
from jax.experimental.pallas import tpu_sc as plsc

_NC = 2
_NS = 16
_B = 80


def _sc_segment_relu_sum(h, ea, src, dst, zrows):
    N, D = h.shape
    E = src.shape[0]
    NW = _NC * _NS
    EW = E // NW
    NCHUNK = EW // _B
    ROWS_T = (N // _NS) // 8 * 8
    REM = N - ROWS_T * _NS
    assert EW * NW == E and NCHUNK * _B == EW and REM % 8 == 0

    mesh = plsc.VectorSubcoreMesh(core_axis_name="c", subcore_axis_name="s")

    @functools.partial(
        pl.kernel,
        out_type=jax.ShapeDtypeStruct((_NC, N, D), jnp.float32),
        mesh=mesh,
        scratch_types=[
            pltpu.VMEM((_B,), jnp.int32),
            pltpu.VMEM((_B,), jnp.int32),
            pltpu.VMEM((_B, D), jnp.float32),
            pltpu.VMEM((_B, D), jnp.float32),
            pltpu.VMEM_SHARED((N, D), jnp.float32),
            pltpu.SemaphoreType.DMA,
            pltpu.SemaphoreType.DMA,
        ],
    )
    def agg_kernel(h_hbm, ea_hbm, src_hbm, dst_hbm, z_hbm, out_hbm,
                   sidx, didx, gbuf, ebuf, acc, sem_g, sem_e):
        cid = lax.axis_index("c")
        sid = lax.axis_index("s")
        wid = sid * _NC + cid

        r0 = sid * ROWS_T
        pltpu.sync_copy(z_hbm.at[pl.ds(r0, ROWS_T)], acc.at[pl.ds(r0, ROWS_T)])

        @pl.when(sid == _NS - 1)
        def _():
            rr = _NS * ROWS_T
            pltpu.sync_copy(z_hbm.at[pl.ds(rr, REM)], acc.at[pl.ds(rr, REM)])

        plsc.subcore_barrier()

        def chunk_body(j, carry):
            base = wid * EW + j * _B
            pltpu.sync_copy(src_hbm.at[pl.ds(base, _B)], sidx)
            pltpu.sync_copy(dst_hbm.at[pl.ds(base, _B)], didx)
            cg = pltpu.async_copy(h_hbm.at[sidx], gbuf, sem_g)
            ce = pltpu.async_copy(ea_hbm.at[pl.ds(base, _B)], ebuf, sem_e)
            cg.wait()
            ce.wait()

            def row_body(r, c):
                for k in range(D // 16):
                    sl = pl.ds(k * 16, 16)
                    gbuf[r, sl] = jnp.maximum(gbuf[r, sl] + ebuf[r, sl], 0.0)
                return c

            lax.fori_loop(0, _B, row_body, 0)
            pltpu.sync_copy(gbuf, acc.at[didx], add=True)
            return carry

        lax.fori_loop(0, NCHUNK, chunk_body, 0)
        plsc.subcore_barrier()

        pltpu.sync_copy(acc.at[pl.ds(r0, ROWS_T)],
                        out_hbm.at[cid, pl.ds(r0, ROWS_T)])

        @pl.when(sid == _NS - 1)
        def _():
            rr = _NS * ROWS_T
            pltpu.sync_copy(acc.at[pl.ds(rr, REM)],
                            out_hbm.at[cid, pl.ds(rr, REM)])

    return agg_kernel(h, ea, src, dst, zrows)


def _ea_body(ea_ref, w1_ref, b1_ref, w2_ref, b2_ref, ea1_ref, ea2_ref):
    a1 = lax.dot_general(ea_ref[...], w1_ref[...],
                         (((1,), (1,)), ((), ())),
                         preferred_element_type=jnp.float32) + b1_ref[...]
    ea1_ref[...] = a1
    ea2_ref[...] = lax.dot_general(a1, w2_ref[...],
                                   (((1,), (1,)), ((), ())),
                                   preferred_element_type=jnp.float32) + b2_ref[...]


def _edge_features(edge_attr, W1, b1, W2, b2):
    E, DE = edge_attr.shape
    D = W1.shape[0]
    H = W2.shape[0]
    BE = 5000
    grid = E // BE
    return pl.pallas_call(
        _ea_body,
        grid=(grid,),
        in_specs=[
            pl.BlockSpec((BE, DE), lambda i: (i, 0)),
            pl.BlockSpec((D, DE), lambda i: (0, 0)),
            pl.BlockSpec((1, D), lambda i: (0, 0)),
            pl.BlockSpec((H, D), lambda i: (0, 0)),
            pl.BlockSpec((1, H), lambda i: (0, 0)),
        ],
        out_specs=[
            pl.BlockSpec((BE, D), lambda i: (i, 0)),
            pl.BlockSpec((BE, H), lambda i: (i, 0)),
        ],
        out_shape=[
            jax.ShapeDtypeStruct((E, D), jnp.float32),
            jax.ShapeDtypeStruct((E, H), jnp.float32),
        ],
    )(edge_attr, W1, b1.reshape(1, D), W2, b2.reshape(1, H))


def _node_body(h_ref, a0_ref, a1_ref, w_ref, b_ref, o_ref):
    s = h_ref[...] + a0_ref[...] + a1_ref[...]
    y = lax.dot_general(s, w_ref[...], (((1,), (1,)), ((), ())),
                        preferred_element_type=jnp.float32) + b_ref[...]
    o_ref[...] = jnp.where(y > 0, y, jnp.exp(y) - 1.0)


def _node_update(h, agg, W, b):
    N, D = h.shape
    H = W.shape[0]
    BN = 1000
    grid = N // BN
    return pl.pallas_call(
        _node_body,
        grid=(grid,),
        in_specs=[
            pl.BlockSpec((BN, D), lambda i: (i, 0)),
            pl.BlockSpec((BN, D), lambda i: (i, 0)),
            pl.BlockSpec((BN, D), lambda i: (i, 0)),
            pl.BlockSpec((H, D), lambda i: (0, 0)),
            pl.BlockSpec((1, H), lambda i: (0, 0)),
        ],
        out_specs=pl.BlockSpec((BN, H), lambda i: (i, 0)),
        out_shape=jax.ShapeDtypeStruct((N, H), jnp.float32),
    )(h, agg[0], agg[1], W, b.reshape(1, H))


def _head_body(h_ref, batch_ref, w1_ref, b1_ref, w2_ref, b2_ref, o_ref):
    G = o_ref.shape[0]
    N = h_ref.shape[0]
    gids = lax.broadcasted_iota(jnp.int32, (G, N), 0)
    onehot = (gids == batch_ref[...]).astype(jnp.float32)
    pooled = lax.dot_general(onehot, h_ref[...], (((1,), (0,)), ((), ())),
                             preferred_element_type=jnp.float32,
                             precision=lax.Precision.HIGHEST)
    p1 = lax.dot_general(pooled, w1_ref[...], (((1,), (1,)), ((), ())),
                         preferred_element_type=jnp.float32) + b1_ref[...]
    o_ref[...] = lax.dot_general(p1, w2_ref[...], (((1,), (1,)), ((), ())),
                                 preferred_element_type=jnp.float32) + b2_ref[...]


def _head(h, batch, Wo1, bo1, Wo2, bo2):
    N, H = h.shape
    G = 64
    C = Wo2.shape[0]
    Wo2p = jnp.zeros((128, H), jnp.float32).at[:C].set(Wo2)
    bo2p = jnp.zeros((1, 128), jnp.float32).at[0, :C].set(bo2)
    out = pl.pallas_call(
        _head_body,
        in_specs=[
            pl.BlockSpec((N, H), lambda: (0, 0)),
            pl.BlockSpec((1, N), lambda: (0, 0)),
            pl.BlockSpec((H, H), lambda: (0, 0)),
            pl.BlockSpec((1, H), lambda: (0, 0)),
            pl.BlockSpec((128, H), lambda: (0, 0)),
            pl.BlockSpec((1, 128), lambda: (0, 0)),
        ],
        out_specs=pl.BlockSpec((G, 128), lambda: (0, 0)),
        out_shape=jax.ShapeDtypeStruct((G, 128), jnp.float32),
    )(h, batch.reshape(1, N), Wo1, bo1.reshape(1, H), Wo2p, bo2p)
    return out[:, :C]


def kernel(x, edge_index, edge_attr, batch,
           W_ec1, b_ec1, W_ec2, b_ec2,
           W0, b0, W1, b1, W2, b2,
           Wo1, bo1, Wo2, bo2):
    N, D = x.shape
    src = edge_index[0]
    dst = edge_index[1]
    zrows = jnp.zeros((N, D), jnp.float32)

    ea1, ea2 = _edge_features(edge_attr, W_ec1, b_ec1, W_ec2, b_ec2)

    agg = _sc_segment_relu_sum(x, ea1, src, dst, zrows)
    h = _node_update(x, agg, W0, b0)
    agg = _sc_segment_relu_sum(h, ea2, src, dst, zrows)
    h = _node_update(h, agg, W1, b1)
    agg = _sc_segment_relu_sum(h, ea2, src, dst, zrows)
    h = _node_update(h, agg, W2, b2)

    return _head(h, batch, Wo1, bo1, Wo2, bo2)
